# baseline (speedup 1.0000x reference)
import functools

import jax
import jax.numpy as jnp
from jax import lax
from jax.experimental import pallas as pl
from jax.experimental.pallas import tpu as pltpu
from jax.experimental.pallas import tpu_sc as plsc

N = 10000
E = 320000
D = 128
ED = 16

NP = 10240
K = 80
NCORES = 2
NSUB = 16
NW = NCORES * NSUB
EPW = E // NW
NCH = EPW // K
IB = 2000
NBLK = EPW // IB
NCHB = IB // K
ROWS_PER_SUB = NP // NSUB
CPB = K
NCOPY = ROWS_PER_SUB // CPB
CN = NP // D


_EB = 12800


def _edge_w_body(ea_ref, w12_ref, b12_ref, w_ref):
    z = jnp.dot(w12_ref[...], ea_ref[...], preferred_element_type=jnp.float32)
    z = z + b12_ref[...]
    w_ref[...] = jnp.maximum(z, 0.0) + jnp.log(1.0 + jnp.exp(-jnp.abs(z)))


def _edge_weights(ea_t, W12t, b12):
    return pl.pallas_call(
        _edge_w_body,
        grid=(E // _EB,),
        in_specs=[
            pl.BlockSpec((ED, _EB), lambda i: (0, i)),
            pl.BlockSpec((2, ED), lambda i: (0, 0)),
            pl.BlockSpec((2, 1), lambda i: (0, 0)),
        ],
        out_specs=pl.BlockSpec((2, _EB), lambda i: (0, i)),
        out_shape=jax.ShapeDtypeStruct((2, E), jnp.float32),
    )(ea_t, W12t, b12)


def _spmm_body(hp_hbm, ei4_hbm, w4_hbm, out_hbm, outc_hbm,
               sidx_v, didx_v, w_v, rows_a, rows_b, c_local, iota_v, acc_sh,
               c_sh, sem_a, sem_b, sem_sa, sem_sb):
    cid = lax.axis_index("c")
    sid = lax.axis_index("s")
    wid = sid * NCORES + cid

    def _zc(r, carry):
        for c in range(D // 16):
            c_local[r, pl.ds(c * 16, 16)] = jnp.zeros((16,), jnp.float32)
            rows_a[r, pl.ds(c * 16, 16)] = jnp.zeros((16,), jnp.float32)
        return carry

    lax.fori_loop(0, CN, _zc, 0)
    for r in range(CN // 16):
        iota_v[pl.ds(r * 16, 16)] = lax.iota(jnp.int32, 16) + (r * 16)
    for b in range(NCOPY):
        pltpu.sync_copy(rows_a, acc_sh.at[pl.ds(sid * ROWS_PER_SUB + b * CPB, CPB)])

    @pl.when(sid == 0)
    def _():
        pltpu.sync_copy(rows_a, c_sh)

    plsc.subcore_barrier()

    def _gather(t, rows, sem):
        pltpu.async_copy(hp_hbm.at[sidx_v.at[t]], rows, sem)

    def _wait_g(rows, sem):
        pltpu.make_async_copy(hp_hbm.at[pl.ds(0, K)], rows, sem).wait()

    def _scale(t, rows):
        for jj in range(K // 16):
            w16 = w_v[t, pl.ds(jj * 16, 16)]
            didx16 = didx_v[t, pl.ds(jj * 16, 16)]
            plsc.addupdate_scatter(
                c_local,
                [lax.shift_right_logical(didx16, 7),
                 lax.bitwise_and(didx16, 127)],
                w16)
            for l in range(16):
                wspl = jnp.full((16,), w16[l], jnp.float32)
                j = jj * 16 + l
                for c in range(D // 16):
                    rows[j, pl.ds(c * 16, 16)] = rows[j, pl.ds(c * 16, 16)] * wspl

    def _ascat(t, rows, sem):
        pltpu.async_copy(rows, acc_sh.at[didx_v.at[t]], sem, add=True)

    def _wait_s(rows, sem):
        pltpu.make_async_copy(rows, acc_sh.at[pl.ds(0, K)], sem).wait()

    def _block(blk, carry):
        wb = wid * NBLK + blk
        pltpu.sync_copy(ei4_hbm.at[0, wb], sidx_v)
        pltpu.sync_copy(ei4_hbm.at[1, wb], didx_v)
        pltpu.sync_copy(w4_hbm.at[wb], w_v)
        _gather(0, rows_a, sem_a)

        def _pair(i, c2):
            t = i * 2
            _gather(t + 1, rows_b, sem_b)
            _wait_g(rows_a, sem_a)
            _scale(t, rows_a)
            _ascat(t, rows_a, sem_sa)
            _wait_g(rows_b, sem_b)
            _scale(t + 1, rows_b)
            _wait_s(rows_a, sem_sa)
            _gather(t + 2, rows_a, sem_a)
            _ascat(t + 1, rows_b, sem_sb)
            _wait_s(rows_b, sem_sb)
            return c2

        lax.fori_loop(0, (NCHB - 1) // 2, _pair, 0)
        _wait_g(rows_a, sem_a)
        _scale(NCHB - 1, rows_a)
        _ascat(NCHB - 1, rows_a, sem_sa)
        _wait_s(rows_a, sem_sa)
        return carry

    lax.fori_loop(0, NBLK, _block, 0)

    pltpu.sync_copy(c_local, c_sh.at[iota_v], add=True)
    plsc.subcore_barrier()

    for b in range(NCOPY):
        r0 = sid * ROWS_PER_SUB + b * CPB
        pltpu.sync_copy(acc_sh.at[pl.ds(r0, CPB)], rows_a)
        pltpu.sync_copy(rows_a, out_hbm.at[cid, pl.ds(r0, CPB)])
    @pl.when(sid == 0)
    def _():
        pltpu.sync_copy(c_sh, c_local)
        pltpu.sync_copy(c_local, outc_hbm.at[pl.ds(cid * CN, CN)])


_spmm = pl.kernel(
    _spmm_body,
    out_type=[
        jax.ShapeDtypeStruct((NCORES, NP, D), jnp.float32),
        jax.ShapeDtypeStruct((NCORES * CN, D), jnp.float32),
    ],
    mesh=plsc.VectorSubcoreMesh(core_axis_name="c", subcore_axis_name="s",
                                num_cores=NCORES, num_subcores=NSUB),
    scratch_types=[
        pltpu.VMEM((NCHB, K), jnp.int32),
        pltpu.VMEM((NCHB, K), jnp.int32),
        pltpu.VMEM((NCHB, K), jnp.float32),
        pltpu.VMEM((K, D), jnp.float32),
        pltpu.VMEM((K, D), jnp.float32),
        pltpu.VMEM((CN, D), jnp.float32),
        pltpu.VMEM((CN,), jnp.int32),
        pltpu.VMEM_SHARED((NP, D), jnp.float32),
        pltpu.VMEM_SHARED((CN, D), jnp.float32),
        pltpu.SemaphoreType.DMA,
        pltpu.SemaphoreType.DMA,
        pltpu.SemaphoreType.DMA,
        pltpu.SemaphoreType.DMA,
    ],
    compiler_params=pltpu.CompilerParams(needs_layout_passes=False),
)


_RB = 512


def _layer_body(final, p_ref, cd_ref, hp_ref, wl_ref, bl_ref, g_ref, be_ref,
                wo_ref, bo_ref, out_ref):
    h = hp_ref[...]
    agg = p_ref[0] + p_ref[1] - cd_ref[...]
    z = jnp.dot(agg, wl_ref[...], preferred_element_type=jnp.float32) + bl_ref[...]
    r = jnp.maximum(z, 0.0)
    mu = jnp.mean(r, axis=-1, keepdims=True)
    dev = r - mu
    var = jnp.mean(dev * dev, axis=-1, keepdims=True)
    ln = dev * lax.rsqrt(var + 1e-5) * g_ref[...] + be_ref[...]
    h2 = ln + h
    if final:
        out_ref[...] = (
            jnp.dot(h2, wo_ref[...], preferred_element_type=jnp.float32) + bo_ref[...]
        )
    else:
        out_ref[...] = h2


def _layer_tc(final, P, cd, hp, W_l, b_l, g, be, W_o, b_o):
    orows = N if final else NP
    return pl.pallas_call(
        functools.partial(_layer_body, final),
        grid=(NP // _RB,),
        in_specs=[
            pl.BlockSpec((NCORES, _RB, D), lambda i: (0, i, 0)),
            pl.BlockSpec((_RB, D), lambda i: (i, 0)),
            pl.BlockSpec((_RB, D), lambda i: (i, 0)),
            pl.BlockSpec((D, D), lambda i: (0, 0)),
            pl.BlockSpec((1, D), lambda i: (0, 0)),
            pl.BlockSpec((1, D), lambda i: (0, 0)),
            pl.BlockSpec((1, D), lambda i: (0, 0)),
            pl.BlockSpec((D, D), lambda i: (0, 0)),
            pl.BlockSpec((1, D), lambda i: (0, 0)),
        ],
        out_specs=pl.BlockSpec((_RB, D), lambda i: (i, 0)),
        out_shape=jax.ShapeDtypeStruct((orows, D), jnp.float32),
    )(P, cd, hp, W_l, b_l, g, be, W_o, b_o)


def kernel(x, edge_index, edge_attr, W_e1, b_e1, W_l1, b_l1, g1, be1,
           W_e2, b_e2, W_l2, b_l2, g2, be2, W_out, b_out):
    ei4 = edge_index.reshape(2, NW * NBLK, NCHB, K)

    W12t = jnp.stack([W_e1[:, 0], W_e2[:, 0]])
    b12 = jnp.stack([b_e1[0], b_e2[0]]).reshape(2, 1)
    w12 = _edge_weights(edge_attr.T, W12t, b12)
    w124 = w12.reshape(2, NW * NBLK, NCHB, K)
    w1 = w124[0]
    w2 = w124[1]

    xp = jnp.zeros((NP, D), jnp.float32).at[:N, :].set(x)

    b_l1r = b_l1.reshape(1, D)
    g1r = g1.reshape(1, D)
    be1r = be1.reshape(1, D)
    b_l2r = b_l2.reshape(1, D)
    g2r = g2.reshape(1, D)
    be2r = be2.reshape(1, D)
    b_or = b_out.reshape(1, D)

    P1, C1 = _spmm(xp, ei4, w1)
    cd1 = (C1[:CN] + C1[CN:]).reshape(NP, 1) * xp
    h1 = _layer_tc(False, P1, cd1, xp, W_l1, b_l1r, g1r, be1r, W_out, b_or)
    P2, C2 = _spmm(h1, ei4, w2)
    cd2 = (C2[:CN] + C2[CN:]).reshape(NP, 1) * h1
    return _layer_tc(True, P2, cd2, h1, W_l2, b_l2r, g2r, be2r, W_out, b_or)


# speedup vs baseline: 1.0696x; 1.0696x over previous
import functools

import jax
import jax.numpy as jnp
from jax import lax
from jax.experimental import pallas as pl
from jax.experimental.pallas import tpu as pltpu
from jax.experimental.pallas import tpu_sc as plsc

N = 10000
E = 320000
D = 128
ED = 16

NP = 10240
K = 80
NCORES = 2
NSUB = 16
NW = NCORES * NSUB
EPW = E // NW
NCH = EPW // K
IB = 2000
NBLK = EPW // IB
NCHB = IB // K
ROWS_PER_SUB = NP // NSUB
CPB = K
NCOPY = ROWS_PER_SUB // CPB
CN = NP // D


_EB = 12800


def _edge_w_body(ea_ref, w12_ref, b12_ref, w_ref):
    z = jnp.dot(w12_ref[...], ea_ref[...], preferred_element_type=jnp.float32)
    z = z + b12_ref[...]
    w_ref[...] = jnp.maximum(z, 0.0) + jnp.log(1.0 + jnp.exp(-jnp.abs(z)))


def _edge_weights(ea_t, W12t, b12):
    return pl.pallas_call(
        _edge_w_body,
        grid=(E // _EB,),
        in_specs=[
            pl.BlockSpec((ED, _EB), lambda i: (0, i)),
            pl.BlockSpec((2, ED), lambda i: (0, 0)),
            pl.BlockSpec((2, 1), lambda i: (0, 0)),
        ],
        out_specs=pl.BlockSpec((2, _EB), lambda i: (0, i)),
        out_shape=jax.ShapeDtypeStruct((2, E), jnp.float32),
    )(ea_t, W12t, b12)


def _spmm_body(hp_hbm, ei4_hbm, w4_hbm, out_hbm, outc_hbm,
               sidx_v, didx_v, w_v, rows_a, rows_b, c_local, iota_v, acc_sh,
               c_sh, sem_a, sem_b, sem_sa, sem_sb):
    cid = lax.axis_index("c")
    sid = lax.axis_index("s")
    wid = sid * NCORES + cid

    def _zc(r, carry):
        for c in range(D // 16):
            c_local[r, pl.ds(c * 16, 16)] = jnp.zeros((16,), jnp.float32)
            rows_a[r, pl.ds(c * 16, 16)] = jnp.zeros((16,), jnp.float32)
        return carry

    lax.fori_loop(0, CN, _zc, 0)
    for r in range(CN // 16):
        iota_v[pl.ds(r * 16, 16)] = lax.iota(jnp.int32, 16) + (r * 16)
    for b in range(NCOPY):
        pltpu.sync_copy(rows_a, acc_sh.at[pl.ds(sid * ROWS_PER_SUB + b * CPB, CPB)])

    @pl.when(sid == 0)
    def _():
        pltpu.sync_copy(rows_a, c_sh)

    plsc.subcore_barrier()

    def _gather(t, rows, sem):
        pltpu.async_copy(hp_hbm.at[sidx_v.at[t]], rows, sem)

    def _wait_g(rows, sem):
        pltpu.make_async_copy(hp_hbm.at[pl.ds(0, K)], rows, sem).wait()

    def _scale(t, rows):
        for jj in range(K // 16):
            w16 = w_v[t, pl.ds(jj * 16, 16)]
            didx16 = didx_v[t, pl.ds(jj * 16, 16)]
            plsc.addupdate_scatter(
                c_local,
                [lax.shift_right_logical(didx16, 7),
                 lax.bitwise_and(didx16, 127)],
                w16)
            for l in range(16):
                wspl = jnp.full((16,), w16[l], jnp.float32)
                j = jj * 16 + l
                for c in range(D // 16):
                    rows[j, pl.ds(c * 16, 16)] = rows[j, pl.ds(c * 16, 16)] * wspl

    def _scat(t, rows):
        pltpu.sync_copy(rows, acc_sh.at[didx_v.at[t]], add=True)

    def _block(blk, carry):
        wb = wid * NBLK + blk
        pltpu.async_copy(ei4_hbm.at[0, wb], sidx_v, sem_sa)
        pltpu.async_copy(ei4_hbm.at[1, wb], didx_v, sem_sa)
        pltpu.async_copy(w4_hbm.at[wb], w_v, sem_sa)
        pltpu.make_async_copy(ei4_hbm.at[0, wb], sidx_v, sem_sa).wait()
        pltpu.make_async_copy(ei4_hbm.at[1, wb], didx_v, sem_sa).wait()
        pltpu.make_async_copy(w4_hbm.at[wb], w_v, sem_sa).wait()
        _gather(0, rows_a, sem_a)

        def _pair(i, c2):
            t = i * 2
            _gather(t + 1, rows_b, sem_b)
            _wait_g(rows_a, sem_a)
            _scale(t, rows_a)
            _scat(t, rows_a)
            _gather(t + 2, rows_a, sem_a)
            _wait_g(rows_b, sem_b)
            _scale(t + 1, rows_b)
            _scat(t + 1, rows_b)
            return c2

        lax.fori_loop(0, (NCHB - 1) // 2, _pair, 0)
        _wait_g(rows_a, sem_a)
        _scale(NCHB - 1, rows_a)
        _scat(NCHB - 1, rows_a)
        return carry

    lax.fori_loop(0, NBLK, _block, 0)

    pltpu.sync_copy(c_local, c_sh.at[iota_v], add=True)
    plsc.subcore_barrier()

    for b in range(NCOPY):
        r0 = sid * ROWS_PER_SUB + b * CPB
        pltpu.sync_copy(acc_sh.at[pl.ds(r0, CPB)], rows_a)
        pltpu.sync_copy(rows_a, out_hbm.at[cid, pl.ds(r0, CPB)])
    @pl.when(sid == 0)
    def _():
        pltpu.sync_copy(c_sh, c_local)
        pltpu.sync_copy(c_local, outc_hbm.at[pl.ds(cid * CN, CN)])


_spmm = pl.kernel(
    _spmm_body,
    out_type=[
        jax.ShapeDtypeStruct((NCORES, NP, D), jnp.float32),
        jax.ShapeDtypeStruct((NCORES * CN, D), jnp.float32),
    ],
    mesh=plsc.VectorSubcoreMesh(core_axis_name="c", subcore_axis_name="s",
                                num_cores=NCORES, num_subcores=NSUB),
    scratch_types=[
        pltpu.VMEM((NCHB, K), jnp.int32),
        pltpu.VMEM((NCHB, K), jnp.int32),
        pltpu.VMEM((NCHB, K), jnp.float32),
        pltpu.VMEM((K, D), jnp.float32),
        pltpu.VMEM((K, D), jnp.float32),
        pltpu.VMEM((CN, D), jnp.float32),
        pltpu.VMEM((CN,), jnp.int32),
        pltpu.VMEM_SHARED((NP, D), jnp.float32),
        pltpu.VMEM_SHARED((CN, D), jnp.float32),
        pltpu.SemaphoreType.DMA,
        pltpu.SemaphoreType.DMA,
        pltpu.SemaphoreType.DMA,
        pltpu.SemaphoreType.DMA,
    ],
    compiler_params=pltpu.CompilerParams(needs_layout_passes=False),
)


_RB = 512


def _layer_body(final, p_ref, cd_ref, hp_ref, wl_ref, bl_ref, g_ref, be_ref,
                wo_ref, bo_ref, out_ref):
    h = hp_ref[...]
    agg = p_ref[0] + p_ref[1] - cd_ref[...]
    z = jnp.dot(agg, wl_ref[...], preferred_element_type=jnp.float32) + bl_ref[...]
    r = jnp.maximum(z, 0.0)
    mu = jnp.mean(r, axis=-1, keepdims=True)
    dev = r - mu
    var = jnp.mean(dev * dev, axis=-1, keepdims=True)
    ln = dev * lax.rsqrt(var + 1e-5) * g_ref[...] + be_ref[...]
    h2 = ln + h
    if final:
        out_ref[...] = (
            jnp.dot(h2, wo_ref[...], preferred_element_type=jnp.float32) + bo_ref[...]
        )
    else:
        out_ref[...] = h2


def _layer_tc(final, P, cd, hp, W_l, b_l, g, be, W_o, b_o):
    orows = N if final else NP
    return pl.pallas_call(
        functools.partial(_layer_body, final),
        grid=(NP // _RB,),
        in_specs=[
            pl.BlockSpec((NCORES, _RB, D), lambda i: (0, i, 0)),
            pl.BlockSpec((_RB, D), lambda i: (i, 0)),
            pl.BlockSpec((_RB, D), lambda i: (i, 0)),
            pl.BlockSpec((D, D), lambda i: (0, 0)),
            pl.BlockSpec((1, D), lambda i: (0, 0)),
            pl.BlockSpec((1, D), lambda i: (0, 0)),
            pl.BlockSpec((1, D), lambda i: (0, 0)),
            pl.BlockSpec((D, D), lambda i: (0, 0)),
            pl.BlockSpec((1, D), lambda i: (0, 0)),
        ],
        out_specs=pl.BlockSpec((_RB, D), lambda i: (i, 0)),
        out_shape=jax.ShapeDtypeStruct((orows, D), jnp.float32),
    )(P, cd, hp, W_l, b_l, g, be, W_o, b_o)


def kernel(x, edge_index, edge_attr, W_e1, b_e1, W_l1, b_l1, g1, be1,
           W_e2, b_e2, W_l2, b_l2, g2, be2, W_out, b_out):
    ei4 = edge_index.reshape(2, NW * NBLK, NCHB, K)

    W12t = jnp.stack([W_e1[:, 0], W_e2[:, 0]])
    b12 = jnp.stack([b_e1[0], b_e2[0]]).reshape(2, 1)
    w12 = _edge_weights(edge_attr.T, W12t, b12)
    w124 = w12.reshape(2, NW * NBLK, NCHB, K)
    w1 = w124[0]
    w2 = w124[1]

    xp = jnp.zeros((NP, D), jnp.float32).at[:N, :].set(x)

    b_l1r = b_l1.reshape(1, D)
    g1r = g1.reshape(1, D)
    be1r = be1.reshape(1, D)
    b_l2r = b_l2.reshape(1, D)
    g2r = g2.reshape(1, D)
    be2r = be2.reshape(1, D)
    b_or = b_out.reshape(1, D)

    P1, C1 = _spmm(xp, ei4, w1)
    cd1 = (C1[:CN] + C1[CN:]).reshape(NP, 1) * xp
    h1 = _layer_tc(False, P1, cd1, xp, W_l1, b_l1r, g1r, be1r, W_out, b_or)
    P2, C2 = _spmm(h1, ei4, w2)
    cd2 = (C2[:CN] + C2[CN:]).reshape(NP, 1) * h1
    return _layer_tc(True, P2, cd2, h1, W_l2, b_l2r, g2r, be2r, W_out, b_or)


# speedup vs baseline: 1.1058x; 1.0338x over previous
import functools

import jax
import jax.numpy as jnp
from jax import lax
from jax.experimental import pallas as pl
from jax.experimental.pallas import tpu as pltpu
from jax.experimental.pallas import tpu_sc as plsc

N = 10000
E = 320000
D = 128
ED = 16

NP = 10240
K = 80
NCORES = 2
NSUB = 16
NW = NCORES * NSUB
EPW = E // NW
NCH = EPW // K
IB = 2000
NBLK = EPW // IB
NCHB = IB // K
ROWS_PER_SUB = NP // NSUB
CPB = K
NCOPY = ROWS_PER_SUB // CPB
CN = NP // D


_EB = 16000


def _edge_w_body(ea_ref, w12_ref, b12_ref, w_ref):
    z = jnp.dot(w12_ref[...], ea_ref[...], preferred_element_type=jnp.float32)
    z = z + b12_ref[...]
    w_ref[...] = jnp.maximum(z, 0.0) + jnp.log(1.0 + jnp.exp(-jnp.abs(z)))


def _edge_weights(ea_t, W12t, b12):
    return pl.pallas_call(
        _edge_w_body,
        grid=(E // _EB,),
        in_specs=[
            pl.BlockSpec((ED, _EB), lambda i: (0, i)),
            pl.BlockSpec((2, ED), lambda i: (0, 0)),
            pl.BlockSpec((2, 1), lambda i: (0, 0)),
        ],
        out_specs=pl.BlockSpec((2, _EB), lambda i: (0, i)),
        out_shape=jax.ShapeDtypeStruct((2, E), jnp.float32),
    )(ea_t, W12t, b12)


def _spmm_body(hp_hbm, ei4_hbm, w4_hbm, out_hbm, outc_hbm,
               sidx_v, didx_v, w_v, rows_a, rows_b, c_local, iota_v, acc_sh,
               c_sh, sem_a, sem_b, sem_sa, sem_sb):
    cid = lax.axis_index("c")
    sid = lax.axis_index("s")
    wid = sid * NCORES + cid

    def _zc(r, carry):
        for c in range(D // 16):
            c_local[r, pl.ds(c * 16, 16)] = jnp.zeros((16,), jnp.float32)
            rows_a[r, pl.ds(c * 16, 16)] = jnp.zeros((16,), jnp.float32)
        return carry

    lax.fori_loop(0, CN, _zc, 0)
    for r in range(CN // 16):
        iota_v[pl.ds(r * 16, 16)] = lax.iota(jnp.int32, 16) + (r * 16)
    for b in range(NCOPY):
        pltpu.sync_copy(rows_a, acc_sh.at[pl.ds(sid * ROWS_PER_SUB + b * CPB, CPB)])

    @pl.when(sid == 0)
    def _():
        pltpu.sync_copy(rows_a, c_sh)

    plsc.subcore_barrier()

    def _gather(t, rows, sem):
        pltpu.async_copy(hp_hbm.at[sidx_v.at[t]], rows, sem)

    def _wait_g(rows, sem):
        pltpu.make_async_copy(hp_hbm.at[pl.ds(0, K)], rows, sem).wait()

    def _scale(t, rows):
        for jj in range(K // 16):
            w16 = w_v[t, pl.ds(jj * 16, 16)]
            didx16 = didx_v[t, pl.ds(jj * 16, 16)]
            plsc.addupdate_scatter(
                c_local,
                [lax.shift_right_logical(didx16, 7),
                 lax.bitwise_and(didx16, 127)],
                w16)
            for l in range(16):
                wspl = jnp.full((16,), w16[l], jnp.float32)
                j = jj * 16 + l
                for c in range(D // 16):
                    rows[j, pl.ds(c * 16, 16)] = rows[j, pl.ds(c * 16, 16)] * wspl

    def _scat(t, rows):
        pltpu.sync_copy(rows, acc_sh.at[didx_v.at[t]], add=True)

    def _block(blk, carry):
        wb = wid * NBLK + blk
        pltpu.async_copy(ei4_hbm.at[0, wb], sidx_v, sem_sa)
        pltpu.async_copy(ei4_hbm.at[1, wb], didx_v, sem_sa)
        pltpu.async_copy(w4_hbm.at[wb], w_v, sem_sa)
        pltpu.make_async_copy(ei4_hbm.at[0, wb], sidx_v, sem_sa).wait()
        pltpu.make_async_copy(ei4_hbm.at[1, wb], didx_v, sem_sa).wait()
        pltpu.make_async_copy(w4_hbm.at[wb], w_v, sem_sa).wait()
        _gather(0, rows_a, sem_a)

        def _pair(i, c2):
            t = i * 2
            _gather(t + 1, rows_b, sem_b)
            _wait_g(rows_a, sem_a)
            _scale(t, rows_a)
            _scat(t, rows_a)
            _gather(t + 2, rows_a, sem_a)
            _wait_g(rows_b, sem_b)
            _scale(t + 1, rows_b)
            _scat(t + 1, rows_b)
            return c2

        lax.fori_loop(0, (NCHB - 1) // 2, _pair, 0)
        _wait_g(rows_a, sem_a)
        _scale(NCHB - 1, rows_a)
        _scat(NCHB - 1, rows_a)
        return carry

    lax.fori_loop(0, NBLK, _block, 0)

    pltpu.sync_copy(c_local, c_sh.at[iota_v], add=True)
    plsc.subcore_barrier()

    for b in range(NCOPY):
        r0 = sid * ROWS_PER_SUB + b * CPB
        pltpu.sync_copy(acc_sh.at[pl.ds(r0, CPB)], rows_a)
        pltpu.sync_copy(rows_a, out_hbm.at[cid, pl.ds(r0, CPB)])
    @pl.when(sid == 0)
    def _():
        pltpu.sync_copy(c_sh, c_local)
        pltpu.sync_copy(c_local, outc_hbm.at[pl.ds(cid * CN, CN)])


_spmm = pl.kernel(
    _spmm_body,
    out_type=[
        jax.ShapeDtypeStruct((NCORES, NP, D), jnp.float32),
        jax.ShapeDtypeStruct((NCORES * CN, D), jnp.float32),
    ],
    mesh=plsc.VectorSubcoreMesh(core_axis_name="c", subcore_axis_name="s",
                                num_cores=NCORES, num_subcores=NSUB),
    scratch_types=[
        pltpu.VMEM((NCHB, K), jnp.int32),
        pltpu.VMEM((NCHB, K), jnp.int32),
        pltpu.VMEM((NCHB, K), jnp.float32),
        pltpu.VMEM((K, D), jnp.float32),
        pltpu.VMEM((K, D), jnp.float32),
        pltpu.VMEM((CN, D), jnp.float32),
        pltpu.VMEM((CN,), jnp.int32),
        pltpu.VMEM_SHARED((NP, D), jnp.float32),
        pltpu.VMEM_SHARED((CN, D), jnp.float32),
        pltpu.SemaphoreType.DMA,
        pltpu.SemaphoreType.DMA,
        pltpu.SemaphoreType.DMA,
        pltpu.SemaphoreType.DMA,
    ],
    compiler_params=pltpu.CompilerParams(needs_layout_passes=False),
)


_RB = 1024


def _layer_body(final, p_ref, cd_ref, hp_ref, wl_ref, bl_ref, g_ref, be_ref,
                wo_ref, bo_ref, out_ref):
    h = hp_ref[...]
    agg = p_ref[0] + p_ref[1] - cd_ref[...]
    z = jnp.dot(agg, wl_ref[...], preferred_element_type=jnp.float32) + bl_ref[...]
    r = jnp.maximum(z, 0.0)
    mu = jnp.mean(r, axis=-1, keepdims=True)
    dev = r - mu
    var = jnp.mean(dev * dev, axis=-1, keepdims=True)
    ln = dev * lax.rsqrt(var + 1e-5) * g_ref[...] + be_ref[...]
    h2 = ln + h
    if final:
        out_ref[...] = (
            jnp.dot(h2, wo_ref[...], preferred_element_type=jnp.float32) + bo_ref[...]
        )
    else:
        out_ref[...] = h2


def _layer_tc(final, P, cd, hp, W_l, b_l, g, be, W_o, b_o):
    orows = N if final else NP
    return pl.pallas_call(
        functools.partial(_layer_body, final),
        grid=(NP // _RB,),
        in_specs=[
            pl.BlockSpec((NCORES, _RB, D), lambda i: (0, i, 0)),
            pl.BlockSpec((_RB, D), lambda i: (i, 0)),
            pl.BlockSpec((_RB, D), lambda i: (i, 0)),
            pl.BlockSpec((D, D), lambda i: (0, 0)),
            pl.BlockSpec((1, D), lambda i: (0, 0)),
            pl.BlockSpec((1, D), lambda i: (0, 0)),
            pl.BlockSpec((1, D), lambda i: (0, 0)),
            pl.BlockSpec((D, D), lambda i: (0, 0)),
            pl.BlockSpec((1, D), lambda i: (0, 0)),
        ],
        out_specs=pl.BlockSpec((_RB, D), lambda i: (i, 0)),
        out_shape=jax.ShapeDtypeStruct((orows, D), jnp.float32),
    )(P, cd, hp, W_l, b_l, g, be, W_o, b_o)


def kernel(x, edge_index, edge_attr, W_e1, b_e1, W_l1, b_l1, g1, be1,
           W_e2, b_e2, W_l2, b_l2, g2, be2, W_out, b_out):
    ei4 = edge_index.reshape(2, NW * NBLK, NCHB, K)

    W12t = jnp.stack([W_e1[:, 0], W_e2[:, 0]])
    b12 = jnp.stack([b_e1[0], b_e2[0]]).reshape(2, 1)
    w12 = _edge_weights(edge_attr.T, W12t, b12)
    w124 = w12.reshape(2, NW * NBLK, NCHB, K)
    w1 = w124[0]
    w2 = w124[1]

    xp = jnp.zeros((NP, D), jnp.float32).at[:N, :].set(x)

    b_l1r = b_l1.reshape(1, D)
    g1r = g1.reshape(1, D)
    be1r = be1.reshape(1, D)
    b_l2r = b_l2.reshape(1, D)
    g2r = g2.reshape(1, D)
    be2r = be2.reshape(1, D)
    b_or = b_out.reshape(1, D)

    P1, C1 = _spmm(xp, ei4, w1)
    cd1 = (C1[:CN] + C1[CN:]).reshape(NP, 1) * xp
    h1 = _layer_tc(False, P1, cd1, xp, W_l1, b_l1r, g1r, be1r, W_out, b_or)
    P2, C2 = _spmm(h1, ei4, w2)
    cd2 = (C2[:CN] + C2[CN:]).reshape(NP, 1) * h1
    return _layer_tc(True, P2, cd2, h1, W_l2, b_l2r, g2r, be2r, W_out, b_or)
